# baseline (speedup 1.0000x reference)
import functools

import jax
import jax.numpy as jnp
from jax import lax
from jax.experimental import pallas as pl
from jax.experimental.pallas import tpu as pltpu
from jax.experimental.pallas import tpu_sc as plsc

_NC = 2
_NS = 16
_NW = _NC * _NS
_B = 128


def _elu(a):
    return jnp.where(a > 0, a, jnp.exp(jnp.minimum(a, 0.0)) - 1.0)


def _sc_mesh():
    return plsc.VectorSubcoreMesh(core_axis_name="c", subcore_axis_name="s")


def _sc_degree(dst3, npad):
    ch = dst3.shape[1]
    slab = npad // _NS

    @functools.partial(
        pl.kernel,
        out_type=jax.ShapeDtypeStruct((_NC, npad, 16), jnp.float32),
        mesh=_sc_mesh(),
        compiler_params=pltpu.CompilerParams(use_tc_tiling_on_sc=False),
        scratch_types=[
            pltpu.VMEM((ch, _B), jnp.int32),
            pltpu.VMEM((_B, 16), jnp.float32),
            pltpu.VMEM((16, 16), jnp.float32),
            pltpu.VMEM_SHARED((npad, 16), jnp.float32),
            pltpu.SemaphoreType.DMA,
        ],
    )
    def deg_kernel(dst_hbm, out_hbm, dst_v, ones_v, zeros_v, acc_sh, dsem):
        c = lax.axis_index("c")
        s = lax.axis_index("s")
        wid = c * _NS + s
        pltpu.sync_copy(dst_hbm.at[wid], dst_v)

        @pl.loop(0, _B)
        def _(r):
            ones_v[r, :] = jnp.full((16,), 1.0, jnp.float32)

        @pl.loop(0, 16)
        def _(r):
            zeros_v[r, :] = jnp.zeros((16,), jnp.float32)

        @pl.loop(0, slab // 16)
        def _(k):
            pltpu.sync_copy(zeros_v, acc_sh.at[pl.ds(s * slab + k * 16, 16)])

        plsc.subcore_barrier()

        nq = 4
        chpad = -(-ch // nq) * nq

        @pl.loop(0, chpad, step=nq)
        def _(g0):
            for b in range(nq):
                j = g0 + b

                @pl.when(j < ch)
                def _():
                    pltpu.async_copy(ones_v, acc_sh.at[dst_v.at[j]], dsem,
                                     add=True)

                jm = j - nq + 1

                @pl.when(jnp.logical_and(0 <= jm, jm < ch))
                def _():
                    pltpu.make_async_copy(ones_v, acc_sh.at[dst_v.at[jm]],
                                          dsem).wait()

        plsc.subcore_barrier()
        pltpu.sync_copy(
            acc_sh.at[pl.ds(s * slab, slab)], out_hbm.at[c, pl.ds(s * slab, slab)]
        )

    return deg_kernel(dst3)


def _sc_propagate(src3, dst3, table, npad, f):
    ch = src3.shape[1]
    slab = npad // _NS

    ng = 6
    nsc = 3
    nbuf = ng + nsc
    chpad = -(-ch // nbuf) * nbuf

    @functools.partial(
        pl.kernel,
        out_type=jax.ShapeDtypeStruct((_NC, npad, f), jnp.float32),
        mesh=_sc_mesh(),
        compiler_params=pltpu.CompilerParams(use_tc_tiling_on_sc=False),
        scratch_types=[
            pltpu.VMEM((ch, _B), jnp.int32),
            pltpu.VMEM((ch, _B), jnp.int32),
            pltpu.VMEM((nbuf, _B, f), jnp.float32),
            pltpu.VMEM((16, f), jnp.float32),
            pltpu.VMEM_SHARED((npad, f), jnp.float32),
            pltpu.SemaphoreType.DMA,
            pltpu.SemaphoreType.DMA,
        ],
    )
    def prop_kernel(src_hbm, dst_hbm, tbl_hbm, out_hbm, src_v, dst_v, rows_v,
                    zeros_v, acc_sh, gsem, ssem):
        c = lax.axis_index("c")
        s = lax.axis_index("s")
        wid = c * _NS + s
        pltpu.sync_copy(src_hbm.at[wid], src_v)
        pltpu.sync_copy(dst_hbm.at[wid], dst_v)

        @pl.loop(0, 16)
        def _(r):
            for g in range(f // 16):
                zeros_v[r, pl.ds(g * 16, 16)] = jnp.zeros((16,), jnp.float32)

        @pl.loop(0, slab // 16)
        def _(k):
            pltpu.sync_copy(zeros_v, acc_sh.at[pl.ds(s * slab + k * 16, 16)])

        plsc.subcore_barrier()

        def gather(j, b):
            pltpu.async_copy(tbl_hbm.at[src_v.at[j]], rows_v.at[b], gsem)

        def gather_wait(j, b):
            pltpu.make_async_copy(tbl_hbm.at[src_v.at[j]], rows_v.at[b],
                                  gsem).wait()

        def scat_start(j, b):
            pltpu.async_copy(rows_v.at[b], acc_sh.at[dst_v.at[j]], ssem,
                             add=True)

        def scat_wait(j, b):
            pltpu.make_async_copy(rows_v.at[b], acc_sh.at[dst_v.at[j]],
                                  ssem).wait()

        for b in range(ng):
            if b < ch:
                gather(b, b)

        @pl.loop(0, chpad, step=nbuf)
        def _(g0):
            for b in range(nbuf):
                j = g0 + b

                @pl.when(j < ch)
                def _():
                    gather_wait(j, b)
                    scat_start(j, b)

                jm = j - nsc

                @pl.when(jnp.logical_and(0 <= jm, jm < ch))
                def _():
                    scat_wait(jm, (b - nsc) % nbuf)

                jp = j + ng

                @pl.when(jp < ch)
                def _():
                    gather(jp, (b + ng) % nbuf)

        plsc.subcore_barrier()
        pltpu.sync_copy(
            acc_sh.at[pl.ds(s * slab, slab)], out_hbm.at[c, pl.ds(s * slab, slab)]
        )

    return prop_kernel(src3, dst3, table)


def _tc_prep(deg2, x_pad, wcat, npad):
    def body(deg_ref, x_ref, w_ref, h_ref, dinv_ref):
        deg = deg_ref[0] + deg_ref[1] + 1.0
        dinv = lax.rsqrt(jnp.maximum(deg, 1.0))
        h = jnp.dot(x_ref[...], w_ref[...], preferred_element_type=jnp.float32)
        h_ref[...] = h * jnp.concatenate([dinv, dinv], axis=1)
        dinv_ref[...] = dinv

    return pl.pallas_call(
        body,
        out_shape=(
            jax.ShapeDtypeStruct((npad, 32), jnp.float32),
            jax.ShapeDtypeStruct((npad, 16), jnp.float32),
        ),
    )(deg2, x_pad, wcat)


def _tc_combine(parts, htil, dinv, bcat, wblk):
    npad, fin = htil.shape
    fout = wblk.shape[1]

    def body(p_ref, h_ref, dinv_ref, b_ref, w_ref, o_ref):
        dinv16 = dinv_ref[...]
        dw = dinv16 if fin == 16 else jnp.concatenate([dinv16, dinv16], axis=1)
        pre = (p_ref[0] + p_ref[1] + h_ref[...]) * dw + b_ref[...]
        act = _elu(pre)
        o_ref[...] = (
            jnp.dot(act, w_ref[...], preferred_element_type=jnp.float32) * dinv16
        )

    return pl.pallas_call(
        body, out_shape=jax.ShapeDtypeStruct((npad, fout), jnp.float32)
    )(parts, htil, dinv, bcat, wblk)


def _tc_fc(parts, htil, dinv, bcat, n, wp3d, wt3d, pfb1, pfW2, pfb2, tfb1):
    npad = htil.shape[0]
    g = 20
    bn = n // g

    def body(p_ref, h_ref, dinv_ref, b_ref, wp_ref, wt_ref, pfb1_ref,
             pfW2_ref, pfb2_ref, tfb1_ref, pred_ref, targ_ref, h3_scr,
             accp, acct):
        i = pl.program_id(0)

        @pl.when(i == 0)
        def _():
            pre = (p_ref[0] + p_ref[1] + h_ref[...]) * dinv_ref[...] + b_ref[...]
            h3_scr[...] = _elu(pre)
            accp[...] = jnp.zeros_like(accp)
            acct[...] = jnp.zeros_like(acct)

        dn = (((0,), (0,)), ((), ()))
        pacc = jnp.zeros((1, 256), jnp.float32)
        tacc = jnp.zeros((1, 256), jnp.float32)
        for f in range(8):
            hp = h3_scr[pl.ds(i * bn, bn), pl.ds(f, 1)]
            ht = h3_scr[pl.ds(i * bn, bn), pl.ds(8 + f, 1)]
            pacc += lax.dot_general(hp, wp_ref[:, f, :], dn,
                                    preferred_element_type=jnp.float32)
            tacc += lax.dot_general(ht, wt_ref[:, f, :], dn,
                                    preferred_element_type=jnp.float32)
        accp[...] += pacc
        acct[...] += tacc

        @pl.when(i == g - 1)
        def _():
            p = _elu(accp[...] + pfb1_ref[...])
            pred_ref[...] = (
                jnp.dot(p, pfW2_ref[...], preferred_element_type=jnp.float32)
                + pfb2_ref[...]
            )
            targ_ref[...] = acct[...] + tfb1_ref[...]

    full = lambda shape: pl.BlockSpec(shape, lambda i: (0,) * len(shape))
    return pl.pallas_call(
        body,
        grid=(g,),
        in_specs=[
            full((2, npad, 16)),
            full((npad, 16)),
            full((npad, 16)),
            full((1, 16)),
            pl.BlockSpec((bn, 8, 256), lambda i: (i, 0, 0)),
            pl.BlockSpec((bn, 8, 256), lambda i: (i, 0, 0)),
            full((1, 256)),
            full((256, 256)),
            full((1, 256)),
            full((1, 256)),
        ],
        out_specs=[full((1, 256)), full((1, 256))],
        out_shape=(
            jax.ShapeDtypeStruct((1, 256), jnp.float32),
            jax.ShapeDtypeStruct((1, 256), jnp.float32),
        ),
        scratch_shapes=[
            pltpu.VMEM((npad, 16), jnp.float32),
            pltpu.VMEM((1, 256), jnp.float32),
            pltpu.VMEM((1, 256), jnp.float32),
        ],
    )(parts, htil, dinv, bcat, wp3d, wt3d, pfb1, pfW2, pfb2, tfb1)


def kernel(x, edge_index, pW1, pb1, pW2, pb2, pW3, pb3, pfW1, pfb1, pfW2, pfb2,
           tW1, tb1, tW2, tb2, tW3, tb3, tfW1, tfb1):
    n, d = x.shape
    e = edge_index.shape[1]

    npad = -(-(n + 1) // (_NS * 16)) * (_NS * 16)
    ch = -(-e // (_NW * _B))
    epad = ch * _NW * _B - e

    src = edge_index[0]
    dst = edge_index[1]
    if epad:
        ar = jnp.arange(epad, dtype=jnp.int32)
        src = jnp.concatenate([src, ar % 64])
        dst = jnp.concatenate([dst, n + ar % (npad - n)])
    src3 = src.reshape(_NW, ch, _B)
    dst3 = dst.reshape(_NW, ch, _B)

    deg2 = _sc_degree(dst3, npad)
    x_pad = jnp.pad(x, ((0, npad - n), (0, 0)))
    wcat1 = jnp.concatenate([pW1, tW1], axis=1)
    h0til, dinv = _tc_prep(deg2, x_pad, wcat1, npad)

    p1 = _sc_propagate(src3, dst3, h0til, npad, 32)
    w2blk = (
        jnp.zeros((32, 16), jnp.float32).at[:16, :8].set(pW2).at[16:, 8:].set(tW2)
    )
    b1cat = jnp.concatenate([pb1, tb1])[None]
    h1til = _tc_combine(p1, h0til, dinv, b1cat, w2blk)

    p2 = _sc_propagate(src3, dst3, h1til, npad, 16)
    w3blk = (
        jnp.zeros((16, 16), jnp.float32).at[:8, :8].set(pW3).at[8:, 8:].set(tW3)
    )
    b2cat = jnp.concatenate([pb2, tb2])[None]
    h2til = _tc_combine(p2, h1til, dinv, b2cat, w3blk)

    p3 = _sc_propagate(src3, dst3, h2til, npad, 16)
    b3cat = jnp.concatenate([pb3, tb3])[None]
    wp3d = pfW1.reshape(n, 8, 256)
    wt3d = tfW1.reshape(n, 8, 256)
    pred, targ = _tc_fc(
        p3, h2til, dinv, b3cat, n, wp3d, wt3d, pfb1[None], pfW2,
        pfb2[None], tfb1[None]
    )
    return pred, targ


# speedup vs baseline: 1.0452x; 1.0452x over previous
import functools

import jax
import jax.numpy as jnp
from jax import lax
from jax.experimental import pallas as pl
from jax.experimental.pallas import tpu as pltpu
from jax.experimental.pallas import tpu_sc as plsc

_NC = 2
_NS = 16
_NW = _NC * _NS
_B = 128


def _elu(a):
    return jnp.where(a > 0, a, jnp.exp(jnp.minimum(a, 0.0)) - 1.0)


def _sc_mesh():
    return plsc.VectorSubcoreMesh(core_axis_name="c", subcore_axis_name="s")


def _fill(ref, rows, f, value):
    val = jnp.full((16,), value, jnp.float32)

    @pl.loop(0, rows, unroll=8)
    def _(r):
        for gi in range(f // 16):
            ref[r, pl.ds(gi * 16, 16)] = val


def _sc_degree(dst3, npad):
    ch = dst3.shape[1]
    slab = npad // _NS

    @functools.partial(
        pl.kernel,
        out_type=jax.ShapeDtypeStruct((_NC, npad, 16), jnp.float32),
        mesh=_sc_mesh(),
        compiler_params=pltpu.CompilerParams(use_tc_tiling_on_sc=False),
        scratch_types=[
            pltpu.VMEM((ch, _B), jnp.int32),
            pltpu.VMEM((_B, 16), jnp.float32),
            pltpu.VMEM((slab, 16), jnp.float32),
            pltpu.VMEM_SHARED((npad, 16), jnp.float32),
            pltpu.SemaphoreType.DMA,
        ],
    )
    def deg_kernel(dst_hbm, out_hbm, dst_v, ones_v, zeros_v, acc_sh, dsem):
        c = lax.axis_index("c")
        s = lax.axis_index("s")
        wid = c * _NS + s
        pltpu.sync_copy(dst_hbm.at[wid], dst_v)
        _fill(ones_v, _B, 16, 1.0)
        _fill(zeros_v, slab, 16, 0.0)
        pltpu.sync_copy(zeros_v, acc_sh.at[pl.ds(s * slab, slab)])
        plsc.subcore_barrier()

        nq = 4
        chpad = -(-ch // nq) * nq

        @pl.loop(0, chpad, step=nq)
        def _(g0):
            for b in range(nq):
                j = g0 + b

                @pl.when(j < ch)
                def _():
                    pltpu.async_copy(ones_v, acc_sh.at[dst_v.at[j]], dsem,
                                     add=True)

                jm = j - nq + 1

                @pl.when(jnp.logical_and(0 <= jm, jm < ch))
                def _():
                    pltpu.make_async_copy(ones_v, acc_sh.at[dst_v.at[jm]],
                                          dsem).wait()

        plsc.subcore_barrier()
        pltpu.sync_copy(
            acc_sh.at[pl.ds(s * slab, slab)], out_hbm.at[c, pl.ds(s * slab, slab)]
        )

    return deg_kernel(dst3)


def _sc_propagate(src3, dst3, table, npad, f):
    ch = src3.shape[1]
    slab = npad // _NS

    ng = 6
    nsc = 3
    nbuf = ng + nsc
    chpad = -(-ch // nbuf) * nbuf

    @functools.partial(
        pl.kernel,
        out_type=jax.ShapeDtypeStruct((_NC, npad, f), jnp.float32),
        mesh=_sc_mesh(),
        compiler_params=pltpu.CompilerParams(use_tc_tiling_on_sc=False),
        scratch_types=[
            pltpu.VMEM((ch, _B), jnp.int32),
            pltpu.VMEM((ch, _B), jnp.int32),
            pltpu.VMEM((nbuf, _B, f), jnp.float32),
            pltpu.VMEM((slab, f), jnp.float32),
            pltpu.VMEM_SHARED((npad, f), jnp.float32),
            pltpu.SemaphoreType.DMA,
            pltpu.SemaphoreType.DMA,
        ],
    )
    def prop_kernel(src_hbm, dst_hbm, tbl_hbm, out_hbm, src_v, dst_v, rows_v,
                    zeros_v, acc_sh, gsem, ssem):
        c = lax.axis_index("c")
        s = lax.axis_index("s")
        wid = c * _NS + s
        pltpu.sync_copy(src_hbm.at[wid], src_v)
        pltpu.sync_copy(dst_hbm.at[wid], dst_v)
        _fill(zeros_v, slab, f, 0.0)
        pltpu.sync_copy(zeros_v, acc_sh.at[pl.ds(s * slab, slab)])
        plsc.subcore_barrier()

        def gather(j, b):
            pltpu.async_copy(tbl_hbm.at[src_v.at[j]], rows_v.at[b], gsem)

        def gather_wait(j, b):
            pltpu.make_async_copy(tbl_hbm.at[src_v.at[j]], rows_v.at[b],
                                  gsem).wait()

        def scat_start(j, b):
            pltpu.async_copy(rows_v.at[b], acc_sh.at[dst_v.at[j]], ssem,
                             add=True)

        def scat_wait(j, b):
            pltpu.make_async_copy(rows_v.at[b], acc_sh.at[dst_v.at[j]],
                                  ssem).wait()

        for b in range(ng):
            if b < ch:
                gather(b, b)

        @pl.loop(0, chpad, step=nbuf)
        def _(g0):
            for b in range(nbuf):
                j = g0 + b

                @pl.when(j < ch)
                def _():
                    gather_wait(j, b)
                    scat_start(j, b)

                jm = j - nsc

                @pl.when(jnp.logical_and(0 <= jm, jm < ch))
                def _():
                    scat_wait(jm, (b - nsc) % nbuf)

                jp = j + ng

                @pl.when(jp < ch)
                def _():
                    gather(jp, (b + ng) % nbuf)

        plsc.subcore_barrier()
        pltpu.sync_copy(
            acc_sh.at[pl.ds(s * slab, slab)], out_hbm.at[c, pl.ds(s * slab, slab)]
        )

    return prop_kernel(src3, dst3, table)


def _tc_prep(deg2, x_pad, wcat, npad):
    def body(deg_ref, x_ref, w_ref, h_ref, dinv_ref):
        deg = deg_ref[0] + deg_ref[1] + 1.0
        dinv = lax.rsqrt(jnp.maximum(deg, 1.0))
        h = jnp.dot(x_ref[...], w_ref[...], preferred_element_type=jnp.float32)
        h_ref[...] = h * jnp.concatenate([dinv, dinv], axis=1)
        dinv_ref[...] = dinv

    return pl.pallas_call(
        body,
        out_shape=(
            jax.ShapeDtypeStruct((npad, 32), jnp.float32),
            jax.ShapeDtypeStruct((npad, 16), jnp.float32),
        ),
    )(deg2, x_pad, wcat)


def _tc_combine(parts, htil, dinv, bcat, wblk):
    npad, fin = htil.shape
    fout = wblk.shape[1]

    def body(p_ref, h_ref, dinv_ref, b_ref, w_ref, o_ref):
        dinv16 = dinv_ref[...]
        dw = dinv16 if fin == 16 else jnp.concatenate([dinv16, dinv16], axis=1)
        pre = (p_ref[0] + p_ref[1] + h_ref[...]) * dw + b_ref[...]
        act = _elu(pre)
        o_ref[...] = (
            jnp.dot(act, w_ref[...], preferred_element_type=jnp.float32) * dinv16
        )

    return pl.pallas_call(
        body, out_shape=jax.ShapeDtypeStruct((npad, fout), jnp.float32)
    )(parts, htil, dinv, bcat, wblk)


def _tc_final(parts, htil, dinv, bcat):
    npad = htil.shape[0]

    def body(p_ref, h_ref, dinv_ref, b_ref, o_ref):
        pre = (p_ref[0] + p_ref[1] + h_ref[...]) * dinv_ref[...] + b_ref[...]
        o_ref[...] = _elu(pre)

    return pl.pallas_call(
        body, out_shape=jax.ShapeDtypeStruct((npad, 16), jnp.float32)
    )(parts, htil, dinv, bcat)


def _tc_fc(hp, ht, pfW1, pfb1, pfW2, pfb2, tfW1, tfb1):
    g, _, bk = hp.shape

    def body(hp_ref, ht_ref, wp_ref, wt_ref, pfb1_ref, pfW2_ref, pfb2_ref,
             tfb1_ref, pred_ref, targ_ref, accp, acct):
        i = pl.program_id(0)

        @pl.when(i == 0)
        def _():
            accp[...] = jnp.zeros_like(accp)
            acct[...] = jnp.zeros_like(acct)

        accp[...] += jnp.dot(
            hp_ref[0], wp_ref[...], preferred_element_type=jnp.float32
        )
        acct[...] += jnp.dot(
            ht_ref[0], wt_ref[...], preferred_element_type=jnp.float32
        )

        @pl.when(i == g - 1)
        def _():
            p = _elu(accp[...] + pfb1_ref[...])
            pred_ref[...] = (
                jnp.dot(p, pfW2_ref[...], preferred_element_type=jnp.float32)
                + pfb2_ref[...]
            )
            targ_ref[...] = acct[...] + tfb1_ref[...]

    full = lambda shape: pl.BlockSpec(shape, lambda i: (0,) * len(shape))
    return pl.pallas_call(
        body,
        grid=(g,),
        in_specs=[
            pl.BlockSpec((1, 1, bk), lambda i: (i, 0, 0)),
            pl.BlockSpec((1, 1, bk), lambda i: (i, 0, 0)),
            pl.BlockSpec((bk, 256), lambda i: (i, 0)),
            pl.BlockSpec((bk, 256), lambda i: (i, 0)),
            full((1, 256)),
            full((256, 256)),
            full((1, 256)),
            full((1, 256)),
        ],
        out_specs=[full((1, 256)), full((1, 256))],
        out_shape=(
            jax.ShapeDtypeStruct((1, 256), jnp.float32),
            jax.ShapeDtypeStruct((1, 256), jnp.float32),
        ),
        scratch_shapes=[
            pltpu.VMEM((1, 256), jnp.float32),
            pltpu.VMEM((1, 256), jnp.float32),
        ],
    )(hp, ht, pfW1, tfW1, pfb1, pfW2, pfb2, tfb1)


def kernel(x, edge_index, pW1, pb1, pW2, pb2, pW3, pb3, pfW1, pfb1, pfW2, pfb2,
           tW1, tb1, tW2, tb2, tW3, tb3, tfW1, tfb1):
    n, d = x.shape
    e = edge_index.shape[1]

    npad = -(-(n + 1) // (_NS * 16)) * (_NS * 16)
    ch = -(-e // (_NW * _B))
    epad = ch * _NW * _B - e

    src = edge_index[0]
    dst = edge_index[1]
    if epad:
        ar = jnp.arange(epad, dtype=jnp.int32)
        src = jnp.concatenate([src, ar % 64])
        dst = jnp.concatenate([dst, n + ar % (npad - n)])
    src3 = src.reshape(_NW, ch, _B)
    dst3 = dst.reshape(_NW, ch, _B)

    deg2 = _sc_degree(dst3, npad)
    x_pad = jnp.pad(x, ((0, npad - n), (0, 0)))
    wcat1 = jnp.concatenate([pW1, tW1], axis=1)
    h0til, dinv = _tc_prep(deg2, x_pad, wcat1, npad)

    p1 = _sc_propagate(src3, dst3, h0til, npad, 32)
    w2blk = (
        jnp.zeros((32, 16), jnp.float32).at[:16, :8].set(pW2).at[16:, 8:].set(tW2)
    )
    b1cat = jnp.concatenate([pb1, tb1])[None]
    h1til = _tc_combine(p1, h0til, dinv, b1cat, w2blk)

    p2 = _sc_propagate(src3, dst3, h1til, npad, 16)
    w3blk = (
        jnp.zeros((16, 16), jnp.float32).at[:8, :8].set(pW3).at[8:, 8:].set(tW3)
    )
    b2cat = jnp.concatenate([pb2, tb2])[None]
    h2til = _tc_combine(p2, h1til, dinv, b2cat, w3blk)

    p3 = _sc_propagate(src3, dst3, h2til, npad, 16)
    b3cat = jnp.concatenate([pb3, tb3])[None]
    h3 = _tc_final(p3, h2til, dinv, b3cat)

    g = 20
    bk = n * 8 // g
    hp = h3[:n, 0:8].reshape(g, 1, bk)
    ht = h3[:n, 8:16].reshape(g, 1, bk)
    pred, targ = _tc_fc(
        hp, ht, pfW1, pfb1[None], pfW2, pfb2[None], tfW1, tfb1[None]
    )
    return pred, targ


# speedup vs baseline: 1.0948x; 1.0475x over previous
import functools

import jax
import jax.numpy as jnp
from jax import lax
from jax.experimental import pallas as pl
from jax.experimental.pallas import tpu as pltpu
from jax.experimental.pallas import tpu_sc as plsc

_NC = 2
_NS = 16
_NW = _NC * _NS
_B = 128


def _elu(a):
    return jnp.where(a > 0, a, jnp.exp(jnp.minimum(a, 0.0)) - 1.0)


def _sc_mesh():
    return plsc.VectorSubcoreMesh(core_axis_name="c", subcore_axis_name="s")


def _fill(ref, rows, f, value):
    val = jnp.full((16,), value, jnp.float32)

    @pl.loop(0, rows, unroll=8)
    def _(r):
        for gi in range(f // 16):
            ref[r, pl.ds(gi * 16, 16)] = val


def _sc_degree(dst3, npad):
    ch = dst3.shape[1]
    slab = npad // _NS

    @functools.partial(
        pl.kernel,
        out_type=jax.ShapeDtypeStruct((_NC, npad, 16), jnp.float32),
        mesh=_sc_mesh(),
        compiler_params=pltpu.CompilerParams(use_tc_tiling_on_sc=False),
        scratch_types=[
            pltpu.VMEM((ch, _B), jnp.int32),
            pltpu.VMEM((_B, 16), jnp.float32),
            pltpu.VMEM((slab, 16), jnp.float32),
            pltpu.VMEM_SHARED((npad, 16), jnp.float32),
            pltpu.SemaphoreType.DMA,
        ],
    )
    def deg_kernel(dst_hbm, out_hbm, dst_v, ones_v, zeros_v, acc_sh, dsem):
        c = lax.axis_index("c")
        s = lax.axis_index("s")
        wid = c * _NS + s
        pltpu.sync_copy(dst_hbm.at[wid], dst_v)
        _fill(ones_v, _B, 16, 1.0)
        _fill(zeros_v, slab, 16, 0.0)
        pltpu.sync_copy(zeros_v, acc_sh.at[pl.ds(s * slab, slab)])
        plsc.subcore_barrier()

        nq = 4
        chpad = -(-ch // nq) * nq

        @pl.loop(0, chpad, step=nq)
        def _(g0):
            for b in range(nq):
                j = g0 + b

                @pl.when(j < ch)
                def _():
                    pltpu.async_copy(ones_v, acc_sh.at[dst_v.at[j]], dsem,
                                     add=True)

                jm = j - nq + 1

                @pl.when(jnp.logical_and(0 <= jm, jm < ch))
                def _():
                    pltpu.make_async_copy(ones_v, acc_sh.at[dst_v.at[jm]],
                                          dsem).wait()

        plsc.subcore_barrier()
        pltpu.sync_copy(
            acc_sh.at[pl.ds(s * slab, slab)], out_hbm.at[c, pl.ds(s * slab, slab)]
        )

    return deg_kernel(dst3)


def _sc_propagate(src3, dst3, table, npad, f):
    ch = src3.shape[1]
    slab = npad // _NS

    ng = 8
    nsc = 4
    nbuf = ng + nsc
    chpad = -(-ch // nbuf) * nbuf

    @functools.partial(
        pl.kernel,
        out_type=jax.ShapeDtypeStruct((_NC, npad, f), jnp.float32),
        mesh=_sc_mesh(),
        compiler_params=pltpu.CompilerParams(use_tc_tiling_on_sc=False),
        scratch_types=[
            pltpu.VMEM((ch, _B), jnp.int32),
            pltpu.VMEM((ch, _B), jnp.int32),
            pltpu.VMEM((nbuf, _B, f), jnp.float32),
            pltpu.VMEM((slab, f), jnp.float32),
            pltpu.VMEM_SHARED((npad, f), jnp.float32),
            pltpu.SemaphoreType.DMA,
            pltpu.SemaphoreType.DMA,
        ],
    )
    def prop_kernel(src_hbm, dst_hbm, tbl_hbm, out_hbm, src_v, dst_v, rows_v,
                    zeros_v, acc_sh, gsem, ssem):
        c = lax.axis_index("c")
        s = lax.axis_index("s")
        wid = c * _NS + s
        pltpu.sync_copy(src_hbm.at[wid], src_v)
        pltpu.sync_copy(dst_hbm.at[wid], dst_v)
        _fill(zeros_v, slab, f, 0.0)
        pltpu.sync_copy(zeros_v, acc_sh.at[pl.ds(s * slab, slab)])
        plsc.subcore_barrier()

        def gather(j, b):
            pltpu.async_copy(tbl_hbm.at[src_v.at[j]], rows_v.at[b], gsem)

        def gather_wait(j, b):
            pltpu.make_async_copy(tbl_hbm.at[src_v.at[j]], rows_v.at[b],
                                  gsem).wait()

        def scat_start(j, b):
            pltpu.async_copy(rows_v.at[b], acc_sh.at[dst_v.at[j]], ssem,
                             add=True)

        def scat_wait(j, b):
            pltpu.make_async_copy(rows_v.at[b], acc_sh.at[dst_v.at[j]],
                                  ssem).wait()

        for b in range(ng):
            if b < ch:
                gather(b, b)

        @pl.loop(0, chpad, step=nbuf)
        def _(g0):
            for b in range(nbuf):
                j = g0 + b

                @pl.when(j < ch)
                def _():
                    gather_wait(j, b)
                    scat_start(j, b)

                jm = j - nsc

                @pl.when(jnp.logical_and(0 <= jm, jm < ch))
                def _():
                    scat_wait(jm, (b - nsc) % nbuf)

                jp = j + ng

                @pl.when(jp < ch)
                def _():
                    gather(jp, (b + ng) % nbuf)

        plsc.subcore_barrier()
        pltpu.sync_copy(
            acc_sh.at[pl.ds(s * slab, slab)], out_hbm.at[c, pl.ds(s * slab, slab)]
        )

    return prop_kernel(src3, dst3, table)


def _tc_prep(deg2, x_pad, wcat, npad):
    def body(deg_ref, x_ref, w_ref, h_ref, dinv_ref):
        deg = deg_ref[0] + deg_ref[1] + 1.0
        dinv = lax.rsqrt(jnp.maximum(deg, 1.0))
        h = jnp.dot(x_ref[...], w_ref[...], preferred_element_type=jnp.float32)
        h_ref[...] = h * jnp.concatenate([dinv, dinv], axis=1)
        dinv_ref[...] = dinv

    return pl.pallas_call(
        body,
        out_shape=(
            jax.ShapeDtypeStruct((npad, 32), jnp.float32),
            jax.ShapeDtypeStruct((npad, 16), jnp.float32),
        ),
    )(deg2, x_pad, wcat)


def _tc_combine(parts, htil, dinv, bcat, wblk):
    npad, fin = htil.shape
    fout = wblk.shape[1]

    def body(p_ref, h_ref, dinv_ref, b_ref, w_ref, o_ref):
        dinv16 = dinv_ref[...]
        dw = dinv16 if fin == 16 else jnp.concatenate([dinv16, dinv16], axis=1)
        pre = (p_ref[0] + p_ref[1] + h_ref[...]) * dw + b_ref[...]
        act = _elu(pre)
        o_ref[...] = (
            jnp.dot(act, w_ref[...], preferred_element_type=jnp.float32) * dinv16
        )

    return pl.pallas_call(
        body, out_shape=jax.ShapeDtypeStruct((npad, fout), jnp.float32)
    )(parts, htil, dinv, bcat, wblk)


def _tc_final(parts, htil, dinv, bcat):
    npad = htil.shape[0]

    def body(p_ref, h_ref, dinv_ref, b_ref, o_ref):
        pre = (p_ref[0] + p_ref[1] + h_ref[...]) * dinv_ref[...] + b_ref[...]
        o_ref[...] = _elu(pre)

    return pl.pallas_call(
        body, out_shape=jax.ShapeDtypeStruct((npad, 16), jnp.float32)
    )(parts, htil, dinv, bcat)


def _tc_fc(hp, ht, pfW1, pfb1, pfW2, pfb2, tfW1, tfb1):
    g, _, bk = hp.shape

    def body(hp_ref, ht_ref, wp_ref, wt_ref, pfb1_ref, pfW2_ref, pfb2_ref,
             tfb1_ref, pred_ref, targ_ref, accp, acct):
        i = pl.program_id(0)

        @pl.when(i == 0)
        def _():
            accp[...] = jnp.zeros_like(accp)
            acct[...] = jnp.zeros_like(acct)

        accp[...] += jnp.dot(
            hp_ref[0], wp_ref[...], preferred_element_type=jnp.float32
        )
        acct[...] += jnp.dot(
            ht_ref[0], wt_ref[...], preferred_element_type=jnp.float32
        )

        @pl.when(i == g - 1)
        def _():
            p = _elu(accp[...] + pfb1_ref[...])
            pred_ref[...] = (
                jnp.dot(p, pfW2_ref[...], preferred_element_type=jnp.float32)
                + pfb2_ref[...]
            )
            targ_ref[...] = acct[...] + tfb1_ref[...]

    full = lambda shape: pl.BlockSpec(shape, lambda i: (0,) * len(shape))
    return pl.pallas_call(
        body,
        grid=(g,),
        in_specs=[
            pl.BlockSpec((1, 1, bk), lambda i: (i, 0, 0)),
            pl.BlockSpec((1, 1, bk), lambda i: (i, 0, 0)),
            pl.BlockSpec((bk, 256), lambda i: (i, 0)),
            pl.BlockSpec((bk, 256), lambda i: (i, 0)),
            full((1, 256)),
            full((256, 256)),
            full((1, 256)),
            full((1, 256)),
        ],
        out_specs=[full((1, 256)), full((1, 256))],
        out_shape=(
            jax.ShapeDtypeStruct((1, 256), jnp.float32),
            jax.ShapeDtypeStruct((1, 256), jnp.float32),
        ),
        scratch_shapes=[
            pltpu.VMEM((1, 256), jnp.float32),
            pltpu.VMEM((1, 256), jnp.float32),
        ],
    )(hp, ht, pfW1, tfW1, pfb1, pfW2, pfb2, tfb1)


def kernel(x, edge_index, pW1, pb1, pW2, pb2, pW3, pb3, pfW1, pfb1, pfW2, pfb2,
           tW1, tb1, tW2, tb2, tW3, tb3, tfW1, tfb1):
    n, d = x.shape
    e = edge_index.shape[1]

    npad = -(-(n + 1) // (_NS * 16)) * (_NS * 16)
    ch = -(-e // (_NW * _B))
    epad = ch * _NW * _B - e

    src = edge_index[0]
    dst = edge_index[1]
    if epad:
        ar = jnp.arange(epad, dtype=jnp.int32)
        src = jnp.concatenate([src, ar % 64])
        dst = jnp.concatenate([dst, n + ar % (npad - n)])
    src3 = src.reshape(_NW, ch, _B)
    dst3 = dst.reshape(_NW, ch, _B)

    deg2 = _sc_degree(dst3, npad)
    x_pad = jnp.pad(x, ((0, npad - n), (0, 0)))
    wcat1 = jnp.concatenate([pW1, tW1], axis=1)
    h0til, dinv = _tc_prep(deg2, x_pad, wcat1, npad)

    p1 = _sc_propagate(src3, dst3, h0til, npad, 32)
    w2blk = (
        jnp.zeros((32, 16), jnp.float32).at[:16, :8].set(pW2).at[16:, 8:].set(tW2)
    )
    b1cat = jnp.concatenate([pb1, tb1])[None]
    h1til = _tc_combine(p1, h0til, dinv, b1cat, w2blk)

    p2 = _sc_propagate(src3, dst3, h1til, npad, 16)
    w3blk = (
        jnp.zeros((16, 16), jnp.float32).at[:8, :8].set(pW3).at[8:, 8:].set(tW3)
    )
    b2cat = jnp.concatenate([pb2, tb2])[None]
    h2til = _tc_combine(p2, h1til, dinv, b2cat, w3blk)

    p3 = _sc_propagate(src3, dst3, h2til, npad, 16)
    b3cat = jnp.concatenate([pb3, tb3])[None]
    h3 = _tc_final(p3, h2til, dinv, b3cat)

    g = 20
    bk = n * 8 // g
    hp = h3[:n, 0:8].reshape(g, 1, bk)
    ht = h3[:n, 8:16].reshape(g, 1, bk)
    pred, targ = _tc_fc(
        hp, ht, pfW1, pfb1[None], pfW2, pfb2[None], tfW1, tfb1[None]
    )
    return pred, targ
